# Initial kernel scaffold; baseline (speedup 1.0000x reference)
import jax
import jax.numpy as jnp
from jax.experimental import pallas as pl


def kernel(obj_vecs, pred_vecs, edges, pred_indicators, W1a, b1a, W1b, b1b, W2a, b2a, W2b, b2b):
    raise NotImplementedError("write your pallas kernel here")


# speedup vs baseline: 7.6222x; 7.6222x over previous
import functools

import jax
import jax.numpy as jnp
from jax import lax
from jax.experimental import pallas as pl
from jax.experimental.pallas import tpu as pltpu
from jax.experimental.pallas import tpu_sc as plsc

F32 = jnp.float32

NC = 2
NS = 16
CHUNK = 128


def _project_tables(obj_flat, w1a_s, w1a_o):
    n, d = obj_flat.shape
    to = 2000
    grid = (n // to,)

    def body(x_ref, ws_ref, wo_ref, us_ref, uo_ref):
        x = x_ref[...]
        us_ref[...] = jnp.dot(x, ws_ref[...], preferred_element_type=F32)
        uo_ref[...] = jnp.dot(x, wo_ref[...], preferred_element_type=F32)

    return pl.pallas_call(
        body,
        grid=grid,
        in_specs=[
            pl.BlockSpec((to, d), lambda i: (i, 0)),
            pl.BlockSpec((d, d), lambda i: (0, 0)),
            pl.BlockSpec((d, d), lambda i: (0, 0)),
        ],
        out_specs=[
            pl.BlockSpec((to, d), lambda i: (i, 0)),
            pl.BlockSpec((to, d), lambda i: (i, 0)),
        ],
        out_shape=[
            jax.ShapeDtypeStruct((n, d), F32),
            jax.ShapeDtypeStruct((n, d), F32),
        ],
    )(obj_flat, w1a_s, w1a_o)


def _sc_gather(us, uo, sidx, oidx):
    bt = sidx.shape[0]
    d = us.shape[1]
    per_w = bt // (NC * NS)
    n_full = per_w // CHUNK
    tail = per_w - n_full * CHUNK

    mesh = plsc.VectorSubcoreMesh(core_axis_name="c", subcore_axis_name="s")

    @functools.partial(
        pl.kernel,
        out_type=[
            jax.ShapeDtypeStruct((bt, d), F32),
            jax.ShapeDtypeStruct((bt, d), F32),
        ],
        mesh=mesh,
        scratch_types=[
            pltpu.VMEM((CHUNK,), jnp.int32),
            pltpu.VMEM((CHUNK,), jnp.int32),
            pltpu.VMEM((CHUNK, d), F32),
            pltpu.VMEM((CHUNK, d), F32),
            pltpu.VMEM((16,), jnp.int32),
            pltpu.VMEM((16,), jnp.int32),
            pltpu.VMEM((16, d), F32),
            pltpu.VMEM((16, d), F32),
        ],
    )
    def gather_kernel(us_hbm, uo_hbm, sidx_hbm, oidx_hbm, gs_hbm, go_hbm,
                      si_v, oi_v, sr_v, or_v, si_t, oi_t, sr_t, or_t):
        w = lax.axis_index("c") * NS + lax.axis_index("s")
        w_base = w * per_w

        def step(i, carry):
            base = pl.multiple_of(w_base + i * CHUNK, 8)
            pltpu.sync_copy(sidx_hbm.at[pl.ds(base, CHUNK)], si_v)
            pltpu.sync_copy(oidx_hbm.at[pl.ds(base, CHUNK)], oi_v)
            pltpu.sync_copy(us_hbm.at[si_v], sr_v)
            pltpu.sync_copy(uo_hbm.at[oi_v], or_v)
            pltpu.sync_copy(sr_v, gs_hbm.at[pl.ds(base, CHUNK)])
            pltpu.sync_copy(or_v, go_hbm.at[pl.ds(base, CHUNK)])
            return carry

        lax.fori_loop(0, n_full, step, 0)
        if tail:
            base = pl.multiple_of(w_base + n_full * CHUNK, 8)
            pltpu.sync_copy(sidx_hbm.at[pl.ds(base, tail)], si_t)
            pltpu.sync_copy(oidx_hbm.at[pl.ds(base, tail)], oi_t)
            pltpu.sync_copy(us_hbm.at[si_t], sr_t)
            pltpu.sync_copy(uo_hbm.at[oi_t], or_t)
            pltpu.sync_copy(sr_t, gs_hbm.at[pl.ds(base, tail)])
            pltpu.sync_copy(or_t, go_hbm.at[pl.ds(base, tail)])

    return gather_kernel(us, uo, sidx, oidx)


def _edge_mlp(gs, go, pred_flat, indf, w1a_p, b1a, w1b, b1b):
    bt, d = pred_flat.shape
    h2p = w1b.shape[1]
    h = b1a.shape[1]
    tt = 2560
    grid = (bt // tt,)

    def body(gs_ref, go_ref, p_ref, ind_ref, wp_ref, ba_ref, wb_ref, bb_ref,
             np_ref, vso_ref, i16_ref):
        pp = jnp.dot(p_ref[...], wp_ref[...], preferred_element_type=F32)
        hh = jnp.maximum(gs_ref[...] + go_ref[...] + pp + ba_ref[...], 0.0)
        t2 = jnp.dot(hh, wb_ref[...], preferred_element_type=F32) + bb_ref[...]
        t2 = jnp.maximum(t2, 0.0)
        ind = ind_ref[...]
        np_ref[...] = t2[:, h:2 * h]
        vso_ref[0] = t2[:, :h] * ind
        vso_ref[1] = t2[:, 2 * h:] * ind
        col0 = jax.lax.broadcasted_iota(jnp.int32, (tt, h), 1) == 0
        i16_ref[...] = jnp.where(col0, ind, 0.0)

    return pl.pallas_call(
        body,
        grid=grid,
        in_specs=[
            pl.BlockSpec((tt, d), lambda i: (i, 0)),
            pl.BlockSpec((tt, d), lambda i: (i, 0)),
            pl.BlockSpec((tt, d), lambda i: (i, 0)),
            pl.BlockSpec((tt, 1), lambda i: (i, 0)),
            pl.BlockSpec((d, h), lambda i: (0, 0)),
            pl.BlockSpec((1, h), lambda i: (0, 0)),
            pl.BlockSpec((h, h2p), lambda i: (0, 0)),
            pl.BlockSpec((1, h2p), lambda i: (0, 0)),
        ],
        out_specs=[
            pl.BlockSpec((tt, h), lambda i: (i, 0)),
            pl.BlockSpec((2, tt, h), lambda i: (0, i, 0)),
            pl.BlockSpec((tt, h), lambda i: (i, 0)),
        ],
        out_shape=[
            jax.ShapeDtypeStruct((bt, h), F32),
            jax.ShapeDtypeStruct((2, bt, h), F32),
            jax.ShapeDtypeStruct((bt, h), F32),
        ],
    )(gs, go, pred_flat, indf, w1a_p, b1a, w1b, b1b)


def _chunks(total, step):
    out = []
    off = 0
    while off < total:
        out.append((off, min(step, total - off)))
        off += step
    return out


def _sc_scatter(vso, i16, soidx, z128, io32, n_obj, t_per_b):
    bt2, h = vso.shape
    bt = bt2 // 2
    per_w = t_per_b // NS
    n_full = per_w // CHUNK
    tail = per_w - n_full * CHUNK
    osl = (n_obj // NS) // 8 * 8
    ow = n_obj - (NS - 1) * osl
    ow = (ow + CHUNK - 1) // CHUNK * CHUNK
    assert (NS - 1) * osl + ow >= n_obj and ow <= n_obj

    mesh = plsc.VectorSubcoreMesh(core_axis_name="c", subcore_axis_name="s")

    @functools.partial(
        pl.kernel,
        out_type=[
            jax.ShapeDtypeStruct((NC * n_obj, h), F32),
            jax.ShapeDtypeStruct((NC * n_obj, h), F32),
        ],
        mesh=mesh,
        scratch_types=[
            pltpu.VMEM_SHARED((n_obj, h), F32),
            pltpu.VMEM((CHUNK,), jnp.int32),
            pltpu.VMEM((CHUNK, h), F32),
        ],
    )
    def scatter_kernel(vso_hbm, i16_hbm, soidx_hbm, z128_hbm,
                       io32_hbm, pooled_hbm, counts_hbm,
                       acc_sh, idx_v, val_v):
        c = lax.axis_index("c")
        s = lax.axis_index("s")
        sbase = pl.multiple_of(
            jnp.minimum(s * osl, n_obj - ow).astype(jnp.int32), 8)
        w_base = c * t_per_b + s * per_w

        def zero_acc():
            pltpu.sync_copy(z128_hbm, val_v)
            for off, ln in _chunks(ow, CHUNK):
                pltpu.sync_copy(io32_hbm.at[pl.ds(sbase + off, ln)], idx_v)
                pltpu.sync_copy(val_v, acc_sh.at[idx_v])

        def scatter_pass(src_hbm, use_side_offset):
            for side in range(2):
                sv = side * bt
                vv = sv if use_side_offset else 0

                def step(i, carry):
                    base = pl.multiple_of(w_base + i * CHUNK, 8)
                    pltpu.sync_copy(soidx_hbm.at[pl.ds(sv + base, CHUNK)],
                                    idx_v)
                    pltpu.sync_copy(src_hbm.at[pl.ds(vv + base, CHUNK)],
                                    val_v)
                    pltpu.sync_copy(val_v, acc_sh.at[idx_v], add=True)
                    return carry

                lax.fori_loop(0, n_full, step, 0)
                if tail:
                    base = pl.multiple_of(w_base + n_full * CHUNK, 8)
                    pltpu.sync_copy(soidx_hbm.at[pl.ds(sv + base, tail)],
                                    idx_v.at[pl.ds(0, tail)])
                    pltpu.sync_copy(src_hbm.at[pl.ds(vv + base, tail)],
                                    val_v.at[pl.ds(0, tail)])
                    pltpu.sync_copy(z128_hbm.at[pl.ds(0, CHUNK - tail)],
                                    val_v.at[pl.ds(tail, CHUNK - tail)])
                    pltpu.sync_copy(val_v, acc_sh.at[idx_v], add=True)

        def drain(out_hbm):
            for off, ln in _chunks(ow, CHUNK):
                pltpu.sync_copy(io32_hbm.at[pl.ds(sbase + off, ln)], idx_v)
                pltpu.sync_copy(acc_sh.at[idx_v], val_v)
                pltpu.sync_copy(
                    val_v.at[pl.ds(0, ln)],
                    out_hbm.at[pl.ds(c * n_obj + sbase + off, ln)])

        zero_acc()
        plsc.subcore_barrier()
        scatter_pass(vso_hbm, use_side_offset=True)
        plsc.subcore_barrier()
        drain(pooled_hbm)
        plsc.subcore_barrier()
        zero_acc()
        plsc.subcore_barrier()
        scatter_pass(i16_hbm, use_side_offset=False)
        plsc.subcore_barrier()
        drain(counts_hbm)

    return scatter_kernel(vso, i16, soidx, z128, io32)


def _obj_mlp(pooled, counts, w2a, b2a, w2b, b2b):
    n, h = pooled.shape
    dout = w2b.shape[1]
    to = 2000
    grid = (n // to,)

    def body(p_ref, c_ref, wa_ref, ba_ref, wb_ref, bb_ref, out_ref):
        cnt = c_ref[...][:, 0:1]
        denom = jnp.where(cnt > 0.0, cnt, 1.0)
        x = p_ref[...] / denom
        hh = jnp.dot(x, wa_ref[...], preferred_element_type=F32) + ba_ref[...]
        hh = jnp.maximum(hh, 0.0)
        y = jnp.dot(hh, wb_ref[...], preferred_element_type=F32) + bb_ref[...]
        out_ref[...] = jnp.maximum(y, 0.0)

    return pl.pallas_call(
        body,
        grid=grid,
        in_specs=[
            pl.BlockSpec((to, h), lambda i: (i, 0)),
            pl.BlockSpec((to, 16), lambda i: (i, 0)),
            pl.BlockSpec((h, h), lambda i: (0, 0)),
            pl.BlockSpec((1, h), lambda i: (0, 0)),
            pl.BlockSpec((h, dout), lambda i: (0, 0)),
            pl.BlockSpec((1, dout), lambda i: (0, 0)),
        ],
        out_specs=pl.BlockSpec((to, dout), lambda i: (i, 0)),
        out_shape=jax.ShapeDtypeStruct((n, dout), F32),
    )(pooled, counts, w2a, b2a, w2b, b2b)


def kernel(obj_vecs, pred_vecs, edges, pred_indicators,
           W1a, b1a, W1b, b1b, W2a, b2a, W2b, b2b):
    b, o, d = obj_vecs.shape
    t = pred_vecs.shape[1]
    h = W1b.shape[0]

    s_idx = edges[:, :, 0]
    o_idx = edges[:, :, 1]
    boff = (jnp.arange(b, dtype=jnp.int32) * o)[:, None]
    sidx_g = (s_idx + boff).reshape(-1)
    oidx_g = (o_idx + boff).reshape(-1)
    sidx = s_idx.reshape(-1)
    oidx = o_idx.reshape(-1)
    indf = pred_indicators.astype(F32).reshape(-1, 1)

    obj_flat = obj_vecs.reshape(b * o, d)
    pred_flat = pred_vecs.reshape(b * t, d)
    w1a_s = W1a[:d]
    w1a_p = W1a[d:2 * d]
    w1a_o = W1a[2 * d:]
    z128 = jnp.zeros((CHUNK, h), F32)
    soidx = jnp.concatenate([sidx, oidx])

    us, uo = _project_tables(obj_flat, w1a_s, w1a_o)
    gs, go = _sc_gather(us, uo, sidx_g, oidx_g)
    new_p, vso, i16 = _edge_mlp(gs, go, pred_flat, indf, w1a_p,
                                b1a.reshape(1, -1), W1b, b1b.reshape(1, -1))
    io32 = jnp.arange(o, dtype=jnp.int32)
    pooled, counts128 = _sc_scatter(vso.reshape(2 * b * t, h), i16, soidx,
                                    z128, io32, o, t)
    counts = counts128[:, :16]
    new_obj = _obj_mlp(pooled, counts, W2a, b2a.reshape(1, -1),
                       W2b, b2b.reshape(1, -1))
    return new_obj.reshape(b, o, -1), new_p.reshape(b, t, -1)
